# baseline (speedup 1.0000x reference)
import functools

import jax
import jax.numpy as jnp
from jax import lax
from jax.experimental import pallas as pl
from jax.experimental.pallas import tpu as pltpu
from jax.experimental.pallas import tpu_sc as plsc

F32 = jnp.float32
N_NODE = 10000
HID = 128
CHUNK = 128
NWORK = 32
NODE_BLK = 2000
EDGE_BLK = 2560
BF16 = jnp.bfloat16


def _tables_body(pos_s_ref, h_ref, x_ref, pos_a_ref, u_ref,
                 wp_ref, wh_ref, wx_ref, bs_ref, wpa_ref, wua_ref,
                 tadst_ref, tssrc_ref, tsdst_ref, tasrc_ref):
    pos_s = pos_s_ref[...]
    wp = wp_ref[...]
    r = (pos_s[:, 0:1] * wp[0:1, :] + pos_s[:, 1:2] * wp[1:2, :]
         + jnp.dot(h_ref[...], wh_ref[...], preferred_element_type=F32)
         + jnp.dot(x_ref[...], wx_ref[...], preferred_element_type=F32)
         + bs_ref[...])
    tadst_ref[...] = r[:, 0:HID]
    tssrc_ref[...] = r[:, HID:2 * HID]
    tsdst_ref[...] = r[:, 2 * HID:3 * HID]
    pos_a = pos_a_ref[...]
    wpa = wpa_ref[...]
    tasrc_ref[...] = (pos_a[:, 0:1] * wpa[0:1, :] + pos_a[:, 1:2] * wpa[1:2, :]
                      + jnp.dot(u_ref[...], wua_ref[...],
                                preferred_element_type=F32))


def _node_tables(pos_s, h, x, pos_a, u, wp, wh, wx, bs, wpa, wua):
    n = pos_s.shape[0]
    grid = (n // NODE_BLK,)
    row = lambda w: pl.BlockSpec((NODE_BLK, w), lambda i: (i, 0))
    full = lambda a, b: pl.BlockSpec((a, b), lambda i: (0, 0))
    return pl.pallas_call(
        _tables_body,
        grid=grid,
        in_specs=[row(2), row(HID), row(HID), row(2), row(HID),
                  full(2, 3 * HID), full(HID, 3 * HID), full(HID, 3 * HID),
                  full(1, 3 * HID), full(2, HID), full(HID, HID)],
        out_specs=[row(HID), row(HID), row(HID), row(HID)],
        out_shape=[jax.ShapeDtypeStruct((n, HID), F32)] * 4,
    )(pos_s, h, x, pos_a, u, wp, wh, wx, bs, wpa, wua)


def _gather4_body(si_a, di_a, si_s, di_s, ta_s, ta_d, ts_s, ts_d,
                  g0, g1, g2, g3, *rest):
    idxv = rest[0:4]
    rowv = rest[4:8]
    sidx, sga, sgs, sta, sts = rest[8:13]
    c = lax.axis_index("c")
    s = lax.axis_index("s")
    wid = s * 2 + c
    per = g0.shape[0] // CHUNK // NWORK
    nw = per * CHUNK
    idx_hbm = (si_a, di_a, si_s, di_s)
    tabs = (ta_s, ta_d, ts_s, ts_d)
    outs = (g0, g1, g2, g3)

    bulk = [pltpu.async_copy(idx_hbm[k].at[pl.ds(wid * nw, nw)], idxv[k],
                             sidx) for k in range(4)]
    for d in bulk:
        d.wait()

    def chunk_ops(k):
        base_e = wid * nw + k * CHUNK
        rows = pl.ds(base_e, CHUNK)
        gA = [pltpu.async_copy(tabs[t].at[idxv[t].at[pl.ds(k * CHUNK, CHUNK)]],
                               rowv[t], sga) for t in (0, 1)]
        gS = [pltpu.async_copy(tabs[t].at[idxv[t].at[pl.ds(k * CHUNK, CHUNK)]],
                               rowv[t], sgs) for t in (2, 3)]
        return rows, gA, gS

    def store_ops(rows):
        stA = [pltpu.async_copy(rowv[t], outs[t].at[rows], sta)
               for t in (0, 1)]
        stS = [pltpu.async_copy(rowv[t], outs[t].at[rows], sts)
               for t in (2, 3)]
        return stA, stS

    rows0, gA, gS = chunk_ops(0)
    for d in gA:
        d.wait()
    for d in gS:
        d.wait()
    st_prev = store_ops(rows0)

    def body(k, carry):
        rows = pl.ds(wid * nw + k * CHUNK, CHUNK)
        for t in (0, 1):
            pltpu.make_async_copy(rowv[t], outs[t].at[rows], sta).wait()
        gA = [pltpu.async_copy(tabs[t].at[idxv[t].at[pl.ds(k * CHUNK, CHUNK)]],
                               rowv[t], sga) for t in (0, 1)]
        for t in (2, 3):
            pltpu.make_async_copy(rowv[t], outs[t].at[rows], sts).wait()
        gS = [pltpu.async_copy(tabs[t].at[idxv[t].at[pl.ds(k * CHUNK, CHUNK)]],
                               rowv[t], sgs) for t in (2, 3)]
        for d in gA:
            d.wait()
        for t in (0, 1):
            pltpu.async_copy(rowv[t], outs[t].at[rows], sta)
        for d in gS:
            d.wait()
        for t in (2, 3):
            pltpu.async_copy(rowv[t], outs[t].at[rows], sts)
        return carry

    lax.fori_loop(1, per, body, 0)
    last = pl.ds(wid * nw + (per - 1) * CHUNK, CHUNK)
    for t in (0, 1):
        pltpu.make_async_copy(rowv[t], outs[t].at[last], sta).wait()
    for t in (2, 3):
        pltpu.make_async_copy(rowv[t], outs[t].at[last], sts).wait()


def _gather4(si_a, di_a, si_s, di_s, ta_s, ta_d, ts_s, ts_d):
    ea = si_a.shape[0]
    mesh = plsc.VectorSubcoreMesh(core_axis_name="c", subcore_axis_name="s")
    per_elems = ea // NWORK
    scratch = ([pltpu.VMEM((per_elems,), jnp.int32)] * 4
               + [pltpu.VMEM((CHUNK, HID), F32)] * 4
               + [pltpu.SemaphoreType.DMA] * 5)
    fn = pl.kernel(
        _gather4_body,
        out_type=[jax.ShapeDtypeStruct((ea, HID), F32)] * 4,
        mesh=mesh,
        scratch_types=scratch,
    )
    return fn(si_a, di_a, si_s, di_s, ta_s, ta_d, ts_s, ts_d)


def _gather2_body(sidx, didx, tsrc, tdst, g0, g1, *rest):
    idxv = rest[0:2]
    rowv = rest[2:6]
    sidx_sem = rest[6]
    sg = rest[7:9]
    st = rest[9:11]
    c = lax.axis_index("c")
    s = lax.axis_index("s")
    wid = s * 2 + c
    per = g0.shape[0] // CHUNK // NWORK
    nw = per * CHUNK
    idx_hbm = (sidx, didx)
    tabs = (tsrc, tdst)
    outs = (g0, g1)

    bulk = [pltpu.async_copy(idx_hbm[t].at[pl.ds(wid * nw, nw)], idxv[t],
                             sidx_sem) for t in range(2)]
    for d in bulk:
        d.wait()

    def fire_gathers(k, p):
        return [pltpu.async_copy(
            tabs[t].at[idxv[t].at[pl.ds(k * CHUNK, CHUNK)]],
            rowv[t * 2 + p], sg[p]) for t in range(2)]

    def fire_stores(k, p):
        rows = pl.ds(wid * nw + k * CHUNK, CHUNK)
        return [pltpu.async_copy(rowv[t * 2 + p], outs[t].at[rows], st[p])
                for t in range(2)]

    def drain_stores(k, p):
        rows = pl.ds(wid * nw + k * CHUNK, CHUNK)
        for t in range(2):
            pltpu.make_async_copy(rowv[t * 2 + p], outs[t].at[rows],
                                  st[p]).wait()

    gds = [fire_gathers(p, p) for p in (0, 1)]
    for p in (0, 1):
        for d in gds[p]:
            d.wait()
        fire_stores(p, p)

    def body(j, carry):
        for p in (0, 1):
            k = 2 * j + p
            drain_stores(k, p)
            gd = fire_gathers(k, p)
            for d in gd:
                d.wait()
            fire_stores(k, p)
        return carry

    lax.fori_loop(1, per // 2, body, 0)
    for p in (0, 1):
        drain_stores(per - 2 + p, p)


def _gather2(sidx, didx, tsrc, tdst):
    ea = sidx.shape[0]
    mesh = plsc.VectorSubcoreMesh(core_axis_name="c", subcore_axis_name="s")
    per_elems = ea // NWORK
    scratch = ([pltpu.VMEM((per_elems,), jnp.int32)] * 2
               + [pltpu.VMEM((CHUNK, HID), F32)] * 4
               + [pltpu.SemaphoreType.DMA] * 5)
    fn = pl.kernel(
        _gather2_body,
        out_type=[jax.ShapeDtypeStruct((ea, HID), F32)] * 2,
        mesh=mesh,
        scratch_types=scratch,
    )
    return fn(sidx, didx, tsrc, tdst)


def _edge_body(gs_ref, gd_ref, dis_ref, wd_ref, w2_ref, b2_ref, w3_ref,
               b3_ref, num_ref, den_ref):
    z1 = jnp.tanh(gs_ref[...] + gd_ref[...] + dis_ref[...] * wd_ref[...])
    h2 = jnp.tanh(jnp.dot(z1, w2_ref[...], preferred_element_type=F32)
                  + b2_ref[...])
    o = jnp.dot(h2, w3_ref[...], preferred_element_type=F32) + b3_ref[...]
    el = jnp.exp(o[:, 0:HID])
    num_ref[...] = el * o[:, HID:2 * HID]
    den_ref[...] = el


def _edge_mlp(gs, gd, dis, wd, w2, b2, w3, b3):
    ea = gs.shape[0]
    grid = (ea // EDGE_BLK,)
    row = lambda w: pl.BlockSpec((EDGE_BLK, w), lambda i: (i, 0))
    full = lambda a, b: pl.BlockSpec((a, b), lambda i: (0, 0))
    return pl.pallas_call(
        _edge_body,
        grid=grid,
        in_specs=[row(HID), row(HID), row(1),
                  full(1, HID), full(HID, HID), full(1, HID),
                  full(HID, 2 * HID), full(1, 2 * HID)],
        out_specs=[row(HID), row(HID)],
        out_shape=[jax.ShapeDtypeStruct((ea, HID), F32)] * 2,
    )(gs, gd, dis, wd, w2, b2, w3, b3)


def _scatter_body(didx, num, den, zeros, out_n, out_d,
                  i0, i1, b0, b1, acc_sh, si0, si1, sl0, sl1, sa0, sa1):
    c = lax.axis_index("c")
    s = lax.axis_index("s")
    n_sub = 16
    per = didx.shape[0] // CHUNK // n_sub
    base_ch = s * per

    @pl.when(s == 0)
    def _():
        pltpu.sync_copy(zeros, acc_sh)

    plsc.subcore_barrier()
    idxs = (i0, i1)
    bufs = (b0, b1)
    sis = (si0, si1)
    sls = (sl0, sl1)
    sas = (sa0, sa1)

    def run(src_hbm):
        def pair(jp, carry):
            ld = []
            for b in range(2):
                base_e = (base_ch + jp * 2 + b) * CHUNK
                ld.append(pltpu.async_copy(didx.at[pl.ds(base_e, CHUNK)],
                                           idxs[b], sis[b]))
                ld.append(pltpu.async_copy(src_hbm.at[pl.ds(base_e, CHUNK)],
                                           bufs[b], sls[b]))
            ad = []
            for b in range(2):
                ld[2 * b].wait()
                ld[2 * b + 1].wait()
                ad.append(pltpu.async_copy(bufs[b], acc_sh.at[idxs[b]],
                                           sas[b], add=True))
            for d in ad:
                d.wait()
            return carry

        lax.fori_loop(0, per // 2, pair, 0)

    pl.when(c == 0)(lambda: run(num))
    pl.when(c == 1)(lambda: run(den))
    plsc.subcore_barrier()

    @pl.when(s < 10)
    def _():
        rows = pl.ds(s * 1000, 1000)
        pl.when(c == 0)(lambda: pltpu.sync_copy(acc_sh.at[rows],
                                                out_n.at[rows]))
        pl.when(c == 1)(lambda: pltpu.sync_copy(acc_sh.at[rows],
                                                out_d.at[rows]))


def _scatter_add(didx, num, den, zeros):
    mesh = plsc.VectorSubcoreMesh(core_axis_name="c", subcore_axis_name="s")
    scratch = [pltpu.VMEM((CHUNK,), jnp.int32),
               pltpu.VMEM((CHUNK,), jnp.int32),
               pltpu.VMEM((CHUNK, HID), F32),
               pltpu.VMEM((CHUNK, HID), F32),
               pltpu.VMEM_SHARED((N_NODE + 8, HID), F32),
               pltpu.SemaphoreType.DMA,
               pltpu.SemaphoreType.DMA,
               pltpu.SemaphoreType.DMA,
               pltpu.SemaphoreType.DMA,
               pltpu.SemaphoreType.DMA,
               pltpu.SemaphoreType.DMA]
    fn = pl.kernel(
        _scatter_body,
        out_type=[jax.ShapeDtypeStruct((N_NODE, HID), F32)] * 2,
        mesh=mesh,
        scratch_types=scratch,
    )
    return fn(didx, num, den, zeros)


def _final_body(pos_ref, h_ref, x_ref, numa_ref, dena_ref, nums_ref, dens_ref,
                wp_ref, wh_ref, wsu_ref, wsx_ref, wx2_ref, b1_ref,
                w2_ref, b2_ref, w3_ref, b3_ref, out_ref):
    dena = dena_ref[...]
    dens = dens_ref[...]
    sum_u = jnp.where(dena != 0, numa_ref[...] / dena, 0.0)
    sum_x = jnp.where(dens != 0, nums_ref[...] / dens, 0.0)
    pos = pos_ref[...]
    wp = wp_ref[...]
    t1 = jnp.tanh(
        pos[:, 0:1] * wp[0:1, :] + pos[:, 1:2] * wp[1:2, :]
        + jnp.dot(h_ref[...], wh_ref[...], preferred_element_type=F32)
        + jnp.dot(sum_u, wsu_ref[...], preferred_element_type=F32)
        + jnp.dot(sum_x, wsx_ref[...], preferred_element_type=F32)
        + jnp.dot(x_ref[...], wx2_ref[...], preferred_element_type=F32)
        + b1_ref[...])
    t2 = jnp.tanh(jnp.dot(t1, w2_ref[...], preferred_element_type=F32)
                  + b2_ref[...])
    out_ref[...] = (jnp.dot(t2, w3_ref[...], preferred_element_type=F32)
                    + b3_ref[...])


def _final_mlp(pos_s, h, x, numa, dena, nums, dens, wp, wh, wsu, wsx, wx2, b1,
               w2, b2, w3, b3):
    n = pos_s.shape[0]
    grid = (n // NODE_BLK,)
    row = lambda w: pl.BlockSpec((NODE_BLK, w), lambda i: (i, 0))
    full = lambda a, b: pl.BlockSpec((a, b), lambda i: (0, 0))
    mlp = 64
    return pl.pallas_call(
        _final_body,
        grid=grid,
        in_specs=[row(2), row(HID), row(HID), row(HID), row(HID), row(HID),
                  row(HID),
                  full(2, mlp), full(HID, mlp), full(HID, mlp),
                  full(HID, mlp), full(HID, mlp), full(1, mlp),
                  full(mlp, mlp), full(1, mlp), full(mlp, HID),
                  full(1, HID)],
        out_specs=row(HID),
        out_shape=jax.ShapeDtypeStruct((n, HID), F32),
    )(pos_s, h, x, numa, dena, nums, dens, wp, wh, wsu, wsx, wx2, b1, w2, b2,
      w3, b3)


def _fuse_heads(pa, pb):
    w1 = jnp.concatenate([pa["W1"], pb["W1"]], axis=1)
    b1 = jnp.concatenate([pa["b1"], pb["b1"]])
    z = jnp.zeros_like(pa["W2"])
    w2 = jnp.concatenate([jnp.concatenate([pa["W2"], z], 1),
                          jnp.concatenate([z, pb["W2"]], 1)], 0)
    b2 = jnp.concatenate([pa["b2"], pb["b2"]])
    z3 = jnp.zeros_like(pa["W3"])
    w3 = jnp.concatenate([jnp.concatenate([pa["W3"], z3], 1),
                          jnp.concatenate([z3, pb["W3"]], 1)], 0)
    b3 = jnp.concatenate([pa["b3"], pb["b3"]])
    return w1, b1, w2, b2, w3, b3


def kernel(h, x, u, pos_state, pos_action, dis_a2s, dis_s2s, edge_a2s,
           edge_s2s, params):
    f = HID
    w1u, b1u, w2u, b2u, w3u, b3u = _fuse_heads(params["u2h_logit"],
                                               params["u2h_u"])
    w1x, b1x, w2x, b2x, w3x, b3x = _fuse_heads(params["x2h_logit"],
                                               params["x2h_x"])
    wp = jnp.concatenate([w1u[2:4], w1x[0:2], w1x[2:4]], axis=1)
    wh = jnp.concatenate([w1u[133:261], w1x[5:133], w1x[261:389]], 1)
    wx = jnp.concatenate([w1u[261:389], w1x[133:261], w1x[389:517]], 1)
    bs = jnp.concatenate([b1u, jnp.zeros_like(b1x), b1x]).reshape(1, 3 * f)
    wpa = w1u[0:2]
    wua = w1u[5:133]
    ta_dst, ts_src, ts_dst, ta_src = _node_tables(
        pos_state, h, x, pos_action, u, wp, wh, wx, bs, wpa, wua)

    ea_raw = edge_a2s.shape[1]
    ea_pad = -(-ea_raw // (NWORK * CHUNK)) * (NWORK * CHUNK)
    pad = ea_pad - ea_raw

    def pad_idx(v, fill):
        return jnp.concatenate(
            [v.astype(jnp.int32), jnp.full((pad,), fill, jnp.int32)])

    src_a2d = pad_idx(edge_a2s[0], 0)
    dst_a2d = pad_idx(edge_a2s[1], N_NODE)
    src_s2d = pad_idx(edge_s2s[0], 0)
    dst_s2d = pad_idx(edge_s2s[1], N_NODE)
    dis_a = jnp.concatenate([dis_a2s, jnp.zeros((pad, 1), F32)])
    dis_s = jnp.concatenate([dis_s2s, jnp.zeros((pad, 1), F32)])

    g_as, g_ad = _gather2(src_a2d, dst_a2d, ta_src, ta_dst)
    g_ss, g_sd = _gather2(src_s2d, dst_s2d, ts_src, ts_dst)

    num_a, den_a = _edge_mlp(g_as, g_ad, dis_a, w1u[4:5], w2u,
                             b2u.reshape(1, 2 * 64), w3u,
                             b3u.reshape(1, 2 * f))
    num_s, den_s = _edge_mlp(g_ss, g_sd, dis_s, w1x[4:5], w2x,
                             b2x.reshape(1, 2 * 64), w3x,
                             b3x.reshape(1, 2 * f))

    zeros = jnp.zeros((N_NODE + 8, f), F32)
    numa, dena = _scatter_add(dst_a2d, num_a, den_a, zeros)
    nums, dens = _scatter_add(dst_s2d, num_s, den_s, zeros)

    pu = params["h_updater"]
    w1f = pu["W1"]
    return _final_mlp(
        pos_state, h, x, numa, dena, nums, dens,
        w1f[0:2], w1f[2:130], w1f[130:258], w1f[258:386], w1f[386:514],
        pu["b1"].reshape(1, -1), pu["W2"], pu["b2"].reshape(1, -1),
        pu["W3"], pu["b3"].reshape(1, -1))


# speedup vs baseline: 1.0502x; 1.0502x over previous
import functools

import jax
import jax.numpy as jnp
from jax import lax
from jax.experimental import pallas as pl
from jax.experimental.pallas import tpu as pltpu
from jax.experimental.pallas import tpu_sc as plsc

F32 = jnp.float32
N_NODE = 10000
HID = 128
CHUNK = 128
NWORK = 32
NODE_BLK = 2000
EDGE_BLK = 2560
BF16 = jnp.bfloat16


def _tables_body(pos_s_ref, h_ref, x_ref, pos_a_ref, u_ref,
                 wp_ref, wh_ref, wx_ref, bs_ref, wpa_ref, wua_ref,
                 tadst_ref, tssrc_ref, tsdst_ref, tasrc_ref):
    pos_s = pos_s_ref[...]
    wp = wp_ref[...]
    r = (pos_s[:, 0:1] * wp[0:1, :] + pos_s[:, 1:2] * wp[1:2, :]
         + jnp.dot(h_ref[...], wh_ref[...], preferred_element_type=F32)
         + jnp.dot(x_ref[...], wx_ref[...], preferred_element_type=F32)
         + bs_ref[...])
    tadst_ref[...] = r[:, 0:HID]
    tssrc_ref[...] = r[:, HID:2 * HID]
    tsdst_ref[...] = r[:, 2 * HID:3 * HID]
    pos_a = pos_a_ref[...]
    wpa = wpa_ref[...]
    tasrc_ref[...] = (pos_a[:, 0:1] * wpa[0:1, :] + pos_a[:, 1:2] * wpa[1:2, :]
                      + jnp.dot(u_ref[...], wua_ref[...],
                                preferred_element_type=F32))


def _node_tables(pos_s, h, x, pos_a, u, wp, wh, wx, bs, wpa, wua):
    n = pos_s.shape[0]
    grid = (n // NODE_BLK,)
    row = lambda w: pl.BlockSpec((NODE_BLK, w), lambda i: (i, 0))
    full = lambda a, b: pl.BlockSpec((a, b), lambda i: (0, 0))
    return pl.pallas_call(
        _tables_body,
        grid=grid,
        in_specs=[row(2), row(HID), row(HID), row(2), row(HID),
                  full(2, 3 * HID), full(HID, 3 * HID), full(HID, 3 * HID),
                  full(1, 3 * HID), full(2, HID), full(HID, HID)],
        out_specs=[row(HID), row(HID), row(HID), row(HID)],
        out_shape=[jax.ShapeDtypeStruct((n, HID), F32)] * 4,
    )(pos_s, h, x, pos_a, u, wp, wh, wx, bs, wpa, wua)


def _gather4_body(si_a, di_a, si_s, di_s, ta_s, ta_d, ts_s, ts_d,
                  g0, g1, g2, g3, *rest):
    idxv = rest[0:4]
    rowv = rest[4:8]
    sems = rest[8:12]
    c = lax.axis_index("c")
    s = lax.axis_index("s")
    wid = s * 2 + c
    per = g0.shape[0] // CHUNK // NWORK
    idx_hbm = (si_a, di_a, si_s, di_s)
    tabs = (ta_s, ta_d, ts_s, ts_d)
    outs = (g0, g1, g2, g3)

    def body(i, carry):
        base = (wid + i * NWORK) * CHUNK
        for k in range(4):
            pltpu.sync_copy(idx_hbm[k].at[pl.ds(base, CHUNK)], idxv[k])
        cps = [pltpu.async_copy(tabs[k].at[idxv[k]], rowv[k], sems[k])
               for k in range(4)]
        for cp in cps:
            cp.wait()
        for k in range(4):
            pltpu.sync_copy(rowv[k], outs[k].at[pl.ds(base, CHUNK)])
        return carry

    lax.fori_loop(0, per, body, 0)


def _gather4(si_a, di_a, si_s, di_s, ta_s, ta_d, ts_s, ts_d):
    ea = si_a.shape[0]
    mesh = plsc.VectorSubcoreMesh(core_axis_name="c", subcore_axis_name="s")
    scratch = ([pltpu.VMEM((CHUNK,), jnp.int32)] * 4
               + [pltpu.VMEM((CHUNK, HID), F32)] * 4
               + [pltpu.SemaphoreType.DMA] * 4)
    fn = pl.kernel(
        _gather4_body,
        out_type=[jax.ShapeDtypeStruct((ea, HID), F32)] * 4,
        mesh=mesh,
        scratch_types=scratch,
    )
    return fn(si_a, di_a, si_s, di_s, ta_s, ta_d, ts_s, ts_d)


def _edge_body(gs_ref, gd_ref, dis_ref, wd_ref, w2_ref, b2_ref, w3_ref,
               b3_ref, num_ref, den_ref):
    z1 = jnp.tanh(gs_ref[...] + gd_ref[...] + dis_ref[...] * wd_ref[...])
    h2 = jnp.tanh(jnp.dot(z1, w2_ref[...], preferred_element_type=F32)
                  + b2_ref[...])
    o = jnp.dot(h2, w3_ref[...], preferred_element_type=F32) + b3_ref[...]
    el = jnp.exp(o[:, 0:HID])
    num_ref[...] = el * o[:, HID:2 * HID]
    den_ref[...] = el


def _edge_mlp(gs, gd, dis, wd, w2, b2, w3, b3):
    ea = gs.shape[0]
    grid = (ea // EDGE_BLK,)
    row = lambda w: pl.BlockSpec((EDGE_BLK, w), lambda i: (i, 0))
    full = lambda a, b: pl.BlockSpec((a, b), lambda i: (0, 0))
    return pl.pallas_call(
        _edge_body,
        grid=grid,
        in_specs=[row(HID), row(HID), row(1),
                  full(1, HID), full(HID, HID), full(1, HID),
                  full(HID, 2 * HID), full(1, 2 * HID)],
        out_specs=[row(HID), row(HID)],
        out_shape=[jax.ShapeDtypeStruct((ea, HID), F32)] * 2,
    )(gs, gd, dis, wd, w2, b2, w3, b3)


def _scatter_body(didx, num, den, zeros, out_n, out_d,
                  i0, i1, b0, b1, acc_sh, si0, si1, sl0, sl1, sa0, sa1):
    c = lax.axis_index("c")
    s = lax.axis_index("s")
    n_sub = 16
    per = didx.shape[0] // CHUNK // n_sub
    base_ch = s * per

    @pl.when(s == 0)
    def _():
        pltpu.sync_copy(zeros, acc_sh)

    plsc.subcore_barrier()
    idxs = (i0, i1)
    bufs = (b0, b1)
    sis = (si0, si1)
    sls = (sl0, sl1)
    sas = (sa0, sa1)

    def run(src_hbm):
        def pair(jp, carry):
            ld = []
            for b in range(2):
                base_e = (base_ch + jp * 2 + b) * CHUNK
                ld.append(pltpu.async_copy(didx.at[pl.ds(base_e, CHUNK)],
                                           idxs[b], sis[b]))
                ld.append(pltpu.async_copy(src_hbm.at[pl.ds(base_e, CHUNK)],
                                           bufs[b], sls[b]))
            ad = []
            for b in range(2):
                ld[2 * b].wait()
                ld[2 * b + 1].wait()
                ad.append(pltpu.async_copy(bufs[b], acc_sh.at[idxs[b]],
                                           sas[b], add=True))
            for d in ad:
                d.wait()
            return carry

        lax.fori_loop(0, per // 2, pair, 0)

    pl.when(c == 0)(lambda: run(num))
    pl.when(c == 1)(lambda: run(den))
    plsc.subcore_barrier()

    @pl.when(s < 10)
    def _():
        rows = pl.ds(s * 1000, 1000)
        pl.when(c == 0)(lambda: pltpu.sync_copy(acc_sh.at[rows],
                                                out_n.at[rows]))
        pl.when(c == 1)(lambda: pltpu.sync_copy(acc_sh.at[rows],
                                                out_d.at[rows]))


def _scatter_add(didx, num, den, zeros):
    mesh = plsc.VectorSubcoreMesh(core_axis_name="c", subcore_axis_name="s")
    scratch = [pltpu.VMEM((CHUNK,), jnp.int32),
               pltpu.VMEM((CHUNK,), jnp.int32),
               pltpu.VMEM((CHUNK, HID), F32),
               pltpu.VMEM((CHUNK, HID), F32),
               pltpu.VMEM_SHARED((N_NODE + 8, HID), F32),
               pltpu.SemaphoreType.DMA,
               pltpu.SemaphoreType.DMA,
               pltpu.SemaphoreType.DMA,
               pltpu.SemaphoreType.DMA,
               pltpu.SemaphoreType.DMA,
               pltpu.SemaphoreType.DMA]
    fn = pl.kernel(
        _scatter_body,
        out_type=[jax.ShapeDtypeStruct((N_NODE, HID), F32)] * 2,
        mesh=mesh,
        scratch_types=scratch,
    )
    return fn(didx, num, den, zeros)


def _final_body(pos_ref, h_ref, x_ref, numa_ref, dena_ref, nums_ref, dens_ref,
                wp_ref, wh_ref, wsu_ref, wsx_ref, wx2_ref, b1_ref,
                w2_ref, b2_ref, w3_ref, b3_ref, out_ref):
    dena = dena_ref[...]
    dens = dens_ref[...]
    sum_u = jnp.where(dena != 0, numa_ref[...] / dena, 0.0)
    sum_x = jnp.where(dens != 0, nums_ref[...] / dens, 0.0)
    pos = pos_ref[...]
    wp = wp_ref[...]
    t1 = jnp.tanh(
        pos[:, 0:1] * wp[0:1, :] + pos[:, 1:2] * wp[1:2, :]
        + jnp.dot(h_ref[...], wh_ref[...], preferred_element_type=F32)
        + jnp.dot(sum_u, wsu_ref[...], preferred_element_type=F32)
        + jnp.dot(sum_x, wsx_ref[...], preferred_element_type=F32)
        + jnp.dot(x_ref[...], wx2_ref[...], preferred_element_type=F32)
        + b1_ref[...])
    t2 = jnp.tanh(jnp.dot(t1, w2_ref[...], preferred_element_type=F32)
                  + b2_ref[...])
    out_ref[...] = (jnp.dot(t2, w3_ref[...], preferred_element_type=F32)
                    + b3_ref[...])


def _final_mlp(pos_s, h, x, numa, dena, nums, dens, wp, wh, wsu, wsx, wx2, b1,
               w2, b2, w3, b3):
    n = pos_s.shape[0]
    grid = (n // NODE_BLK,)
    row = lambda w: pl.BlockSpec((NODE_BLK, w), lambda i: (i, 0))
    full = lambda a, b: pl.BlockSpec((a, b), lambda i: (0, 0))
    mlp = 64
    return pl.pallas_call(
        _final_body,
        grid=grid,
        in_specs=[row(2), row(HID), row(HID), row(HID), row(HID), row(HID),
                  row(HID),
                  full(2, mlp), full(HID, mlp), full(HID, mlp),
                  full(HID, mlp), full(HID, mlp), full(1, mlp),
                  full(mlp, mlp), full(1, mlp), full(mlp, HID),
                  full(1, HID)],
        out_specs=row(HID),
        out_shape=jax.ShapeDtypeStruct((n, HID), F32),
    )(pos_s, h, x, numa, dena, nums, dens, wp, wh, wsu, wsx, wx2, b1, w2, b2,
      w3, b3)


def _fuse_heads(pa, pb):
    w1 = jnp.concatenate([pa["W1"], pb["W1"]], axis=1)
    b1 = jnp.concatenate([pa["b1"], pb["b1"]])
    z = jnp.zeros_like(pa["W2"])
    w2 = jnp.concatenate([jnp.concatenate([pa["W2"], z], 1),
                          jnp.concatenate([z, pb["W2"]], 1)], 0)
    b2 = jnp.concatenate([pa["b2"], pb["b2"]])
    z3 = jnp.zeros_like(pa["W3"])
    w3 = jnp.concatenate([jnp.concatenate([pa["W3"], z3], 1),
                          jnp.concatenate([z3, pb["W3"]], 1)], 0)
    b3 = jnp.concatenate([pa["b3"], pb["b3"]])
    return w1, b1, w2, b2, w3, b3


def kernel(h, x, u, pos_state, pos_action, dis_a2s, dis_s2s, edge_a2s,
           edge_s2s, params):
    f = HID
    w1u, b1u, w2u, b2u, w3u, b3u = _fuse_heads(params["u2h_logit"],
                                               params["u2h_u"])
    w1x, b1x, w2x, b2x, w3x, b3x = _fuse_heads(params["x2h_logit"],
                                               params["x2h_x"])
    wp = jnp.concatenate([w1u[2:4], w1x[0:2], w1x[2:4]], axis=1)
    wh = jnp.concatenate([w1u[133:261], w1x[5:133], w1x[261:389]], 1)
    wx = jnp.concatenate([w1u[261:389], w1x[133:261], w1x[389:517]], 1)
    bs = jnp.concatenate([b1u, jnp.zeros_like(b1x), b1x]).reshape(1, 3 * f)
    wpa = w1u[0:2]
    wua = w1u[5:133]
    ta_dst, ts_src, ts_dst, ta_src = _node_tables(
        pos_state, h, x, pos_action, u, wp, wh, wx, bs, wpa, wua)

    ea_raw = edge_a2s.shape[1]
    ea_pad = -(-ea_raw // (NWORK * CHUNK)) * (NWORK * CHUNK)
    pad = ea_pad - ea_raw

    def pad_idx(v, fill):
        return jnp.concatenate(
            [v.astype(jnp.int32), jnp.full((pad,), fill, jnp.int32)])

    src_a2d = pad_idx(edge_a2s[0], 0)
    dst_a2d = pad_idx(edge_a2s[1], N_NODE)
    src_s2d = pad_idx(edge_s2s[0], 0)
    dst_s2d = pad_idx(edge_s2s[1], N_NODE)
    dis_a = jnp.concatenate([dis_a2s, jnp.zeros((pad, 1), F32)])
    dis_s = jnp.concatenate([dis_s2s, jnp.zeros((pad, 1), F32)])

    g_as, g_ad, g_ss, g_sd = _gather4(
        src_a2d, dst_a2d, src_s2d, dst_s2d, ta_src, ta_dst, ts_src, ts_dst)

    num_a, den_a = _edge_mlp(g_as, g_ad, dis_a, w1u[4:5], w2u,
                             b2u.reshape(1, 2 * 64), w3u,
                             b3u.reshape(1, 2 * f))
    num_s, den_s = _edge_mlp(g_ss, g_sd, dis_s, w1x[4:5], w2x,
                             b2x.reshape(1, 2 * 64), w3x,
                             b3x.reshape(1, 2 * f))

    zeros = jnp.zeros((N_NODE + 8, f), F32)
    numa, dena = _scatter_add(dst_a2d, num_a, den_a, zeros)
    nums, dens = _scatter_add(dst_s2d, num_s, den_s, zeros)

    pu = params["h_updater"]
    w1f = pu["W1"]
    return _final_mlp(
        pos_state, h, x, numa, dena, nums, dens,
        w1f[0:2], w1f[2:130], w1f[130:258], w1f[258:386], w1f[386:514],
        pu["b1"].reshape(1, -1), pu["W2"], pu["b2"].reshape(1, -1),
        pu["W3"], pu["b3"].reshape(1, -1))


# speedup vs baseline: 1.0519x; 1.0017x over previous
import functools

import jax
import jax.numpy as jnp
from jax import lax
from jax.experimental import pallas as pl
from jax.experimental.pallas import tpu as pltpu
from jax.experimental.pallas import tpu_sc as plsc

F32 = jnp.float32
N_NODE = 10000
HID = 128
CHUNK = 128
NWORK = 32
NODE_BLK = 2000
EDGE_BLK = 2560
BF16 = jnp.bfloat16


def _tables_body(pos_s_ref, h_ref, x_ref, pos_a_ref, u_ref,
                 wp_ref, wh_ref, wx_ref, bs_ref, wpa_ref, wua_ref,
                 tadst_ref, tssrc_ref, tsdst_ref, tasrc_ref):
    pos_s = pos_s_ref[...]
    wp = wp_ref[...]
    r = (pos_s[:, 0:1] * wp[0:1, :] + pos_s[:, 1:2] * wp[1:2, :]
         + jnp.dot(h_ref[...], wh_ref[...], preferred_element_type=F32)
         + jnp.dot(x_ref[...], wx_ref[...], preferred_element_type=F32)
         + bs_ref[...])
    tadst_ref[...] = r[:, 0:HID]
    tssrc_ref[...] = r[:, HID:2 * HID]
    tsdst_ref[...] = r[:, 2 * HID:3 * HID]
    pos_a = pos_a_ref[...]
    wpa = wpa_ref[...]
    tasrc_ref[...] = (pos_a[:, 0:1] * wpa[0:1, :] + pos_a[:, 1:2] * wpa[1:2, :]
                      + jnp.dot(u_ref[...], wua_ref[...],
                                preferred_element_type=F32))


def _node_tables(pos_s, h, x, pos_a, u, wp, wh, wx, bs, wpa, wua):
    n = pos_s.shape[0]
    grid = (n // NODE_BLK,)
    row = lambda w: pl.BlockSpec((NODE_BLK, w), lambda i: (i, 0))
    full = lambda a, b: pl.BlockSpec((a, b), lambda i: (0, 0))
    return pl.pallas_call(
        _tables_body,
        grid=grid,
        in_specs=[row(2), row(HID), row(HID), row(2), row(HID),
                  full(2, 3 * HID), full(HID, 3 * HID), full(HID, 3 * HID),
                  full(1, 3 * HID), full(2, HID), full(HID, HID)],
        out_specs=[row(HID), row(HID), row(HID), row(HID)],
        out_shape=[jax.ShapeDtypeStruct((n, HID), F32)] * 4,
    )(pos_s, h, x, pos_a, u, wp, wh, wx, bs, wpa, wua)


def _gather4_body(si_a, di_a, si_s, di_s, ta_s, ta_d, ts_s, ts_d,
                  g0, g1, g2, g3, *rest):
    idxv = rest[0:4]
    rowv = rest[4:8]
    sems = rest[8:12]
    c = lax.axis_index("c")
    s = lax.axis_index("s")
    wid = s * 2 + c
    per = g0.shape[0] // CHUNK // NWORK
    idx_hbm = (si_a, di_a, si_s, di_s)
    tabs = (ta_s, ta_d, ts_s, ts_d)
    outs = (g0, g1, g2, g3)

    def body(i, carry):
        base = (wid + i * NWORK) * CHUNK
        for k in range(4):
            pltpu.sync_copy(idx_hbm[k].at[pl.ds(base, CHUNK)], idxv[k])
        cps = [pltpu.async_copy(tabs[k].at[idxv[k]], rowv[k], sems[k])
               for k in range(4)]
        for cp in cps:
            cp.wait()
        for k in range(4):
            pltpu.sync_copy(rowv[k], outs[k].at[pl.ds(base, CHUNK)])
        return carry

    lax.fori_loop(0, per, body, 0)


def _gather4(si_a, di_a, si_s, di_s, ta_s, ta_d, ts_s, ts_d):
    ea = si_a.shape[0]
    mesh = plsc.VectorSubcoreMesh(core_axis_name="c", subcore_axis_name="s")
    scratch = ([pltpu.VMEM((CHUNK,), jnp.int32)] * 4
               + [pltpu.VMEM((CHUNK, HID), F32)] * 4
               + [pltpu.SemaphoreType.DMA] * 4)
    fn = pl.kernel(
        _gather4_body,
        out_type=[jax.ShapeDtypeStruct((ea, HID), F32)] * 4,
        mesh=mesh,
        scratch_types=scratch,
    )
    return fn(si_a, di_a, si_s, di_s, ta_s, ta_d, ts_s, ts_d)


def _edge_body(gs_ref, gd_ref, dis_ref, wd_ref, w2_ref, b2_ref, w3_ref,
               b3_ref, num_ref, den_ref):
    z1 = jnp.tanh(gs_ref[...] + gd_ref[...] + dis_ref[...] * wd_ref[...])
    h2 = jnp.tanh(jnp.dot(z1, w2_ref[...], preferred_element_type=F32)
                  + b2_ref[...])
    o = jnp.dot(h2, w3_ref[...], preferred_element_type=F32) + b3_ref[...]
    el = jnp.exp(o[:, 0:HID])
    num_ref[...] = el * o[:, HID:2 * HID]
    den_ref[...] = el


def _edge_mlp(gs, gd, dis, wd, w2, b2, w3, b3):
    ea = gs.shape[0]
    grid = (ea // EDGE_BLK,)
    row = lambda w: pl.BlockSpec((EDGE_BLK, w), lambda i: (i, 0))
    full = lambda a, b: pl.BlockSpec((a, b), lambda i: (0, 0))
    return pl.pallas_call(
        _edge_body,
        grid=grid,
        in_specs=[row(HID), row(HID), row(1),
                  full(1, HID), full(HID, HID), full(1, HID),
                  full(HID, 2 * HID), full(1, 2 * HID)],
        out_specs=[row(HID), row(HID)],
        out_shape=[jax.ShapeDtypeStruct((ea, HID), F32)] * 2,
    )(gs, gd, dis, wd, w2, b2, w3, b3)


def _scatter_body(didx, num, den, zeros, out_n, out_d,
                  i0, i1, b0, b1, acc_sh, si0, si1, sl0, sl1, sa0, sa1):
    c = lax.axis_index("c")
    s = lax.axis_index("s")
    n_sub = 16
    per = didx.shape[0] // CHUNK // n_sub

    @pl.when(s == 0)
    def _():
        pltpu.sync_copy(zeros, acc_sh)

    plsc.subcore_barrier()
    idxs = (i0, i1)
    bufs = (b0, b1)
    sis = (si0, si1)
    sls = (sl0, sl1)
    sas = (sa0, sa1)

    def run(src_hbm):
        def pair(jp, carry):
            ld = []
            for b in range(2):
                base_e = (s + (jp * 2 + b) * n_sub) * CHUNK
                ld.append(pltpu.async_copy(didx.at[pl.ds(base_e, CHUNK)],
                                           idxs[b], sis[b]))
                ld.append(pltpu.async_copy(src_hbm.at[pl.ds(base_e, CHUNK)],
                                           bufs[b], sls[b]))
            ad = []
            for b in range(2):
                ld[2 * b].wait()
                ld[2 * b + 1].wait()
                ad.append(pltpu.async_copy(bufs[b], acc_sh.at[idxs[b]],
                                           sas[b], add=True))
            for d in ad:
                d.wait()
            return carry

        lax.fori_loop(0, per // 2, pair, 0)

    pl.when(c == 0)(lambda: run(num))
    pl.when(c == 1)(lambda: run(den))
    plsc.subcore_barrier()

    @pl.when(s < 10)
    def _():
        rows = pl.ds(s * 1000, 1000)
        pl.when(c == 0)(lambda: pltpu.sync_copy(acc_sh.at[rows],
                                                out_n.at[rows]))
        pl.when(c == 1)(lambda: pltpu.sync_copy(acc_sh.at[rows],
                                                out_d.at[rows]))


def _scatter_add(didx, num, den, zeros):
    mesh = plsc.VectorSubcoreMesh(core_axis_name="c", subcore_axis_name="s")
    scratch = [pltpu.VMEM((CHUNK,), jnp.int32),
               pltpu.VMEM((CHUNK,), jnp.int32),
               pltpu.VMEM((CHUNK, HID), F32),
               pltpu.VMEM((CHUNK, HID), F32),
               pltpu.VMEM_SHARED((N_NODE + 8, HID), F32),
               pltpu.SemaphoreType.DMA,
               pltpu.SemaphoreType.DMA,
               pltpu.SemaphoreType.DMA,
               pltpu.SemaphoreType.DMA,
               pltpu.SemaphoreType.DMA,
               pltpu.SemaphoreType.DMA]
    fn = pl.kernel(
        _scatter_body,
        out_type=[jax.ShapeDtypeStruct((N_NODE, HID), F32)] * 2,
        mesh=mesh,
        scratch_types=scratch,
    )
    return fn(didx, num, den, zeros)


def _final_body(pos_ref, h_ref, x_ref, numa_ref, dena_ref, nums_ref, dens_ref,
                wp_ref, wh_ref, wsu_ref, wsx_ref, wx2_ref, b1_ref,
                w2_ref, b2_ref, w3_ref, b3_ref, out_ref):
    dena = dena_ref[...]
    dens = dens_ref[...]
    sum_u = jnp.where(dena != 0, numa_ref[...] / dena, 0.0)
    sum_x = jnp.where(dens != 0, nums_ref[...] / dens, 0.0)
    pos = pos_ref[...]
    wp = wp_ref[...]
    t1 = jnp.tanh(
        pos[:, 0:1] * wp[0:1, :] + pos[:, 1:2] * wp[1:2, :]
        + jnp.dot(h_ref[...], wh_ref[...], preferred_element_type=F32)
        + jnp.dot(sum_u, wsu_ref[...], preferred_element_type=F32)
        + jnp.dot(sum_x, wsx_ref[...], preferred_element_type=F32)
        + jnp.dot(x_ref[...], wx2_ref[...], preferred_element_type=F32)
        + b1_ref[...])
    t2 = jnp.tanh(jnp.dot(t1, w2_ref[...], preferred_element_type=F32)
                  + b2_ref[...])
    out_ref[...] = (jnp.dot(t2, w3_ref[...], preferred_element_type=F32)
                    + b3_ref[...])


def _final_mlp(pos_s, h, x, numa, dena, nums, dens, wp, wh, wsu, wsx, wx2, b1,
               w2, b2, w3, b3):
    n = pos_s.shape[0]
    grid = (n // NODE_BLK,)
    row = lambda w: pl.BlockSpec((NODE_BLK, w), lambda i: (i, 0))
    full = lambda a, b: pl.BlockSpec((a, b), lambda i: (0, 0))
    mlp = 64
    return pl.pallas_call(
        _final_body,
        grid=grid,
        in_specs=[row(2), row(HID), row(HID), row(HID), row(HID), row(HID),
                  row(HID),
                  full(2, mlp), full(HID, mlp), full(HID, mlp),
                  full(HID, mlp), full(HID, mlp), full(1, mlp),
                  full(mlp, mlp), full(1, mlp), full(mlp, HID),
                  full(1, HID)],
        out_specs=row(HID),
        out_shape=jax.ShapeDtypeStruct((n, HID), F32),
    )(pos_s, h, x, numa, dena, nums, dens, wp, wh, wsu, wsx, wx2, b1, w2, b2,
      w3, b3)


def _fuse_heads(pa, pb):
    w1 = jnp.concatenate([pa["W1"], pb["W1"]], axis=1)
    b1 = jnp.concatenate([pa["b1"], pb["b1"]])
    z = jnp.zeros_like(pa["W2"])
    w2 = jnp.concatenate([jnp.concatenate([pa["W2"], z], 1),
                          jnp.concatenate([z, pb["W2"]], 1)], 0)
    b2 = jnp.concatenate([pa["b2"], pb["b2"]])
    z3 = jnp.zeros_like(pa["W3"])
    w3 = jnp.concatenate([jnp.concatenate([pa["W3"], z3], 1),
                          jnp.concatenate([z3, pb["W3"]], 1)], 0)
    b3 = jnp.concatenate([pa["b3"], pb["b3"]])
    return w1, b1, w2, b2, w3, b3


def kernel(h, x, u, pos_state, pos_action, dis_a2s, dis_s2s, edge_a2s,
           edge_s2s, params):
    f = HID
    w1u, b1u, w2u, b2u, w3u, b3u = _fuse_heads(params["u2h_logit"],
                                               params["u2h_u"])
    w1x, b1x, w2x, b2x, w3x, b3x = _fuse_heads(params["x2h_logit"],
                                               params["x2h_x"])
    wp = jnp.concatenate([w1u[2:4], w1x[0:2], w1x[2:4]], axis=1)
    wh = jnp.concatenate([w1u[133:261], w1x[5:133], w1x[261:389]], 1)
    wx = jnp.concatenate([w1u[261:389], w1x[133:261], w1x[389:517]], 1)
    bs = jnp.concatenate([b1u, jnp.zeros_like(b1x), b1x]).reshape(1, 3 * f)
    wpa = w1u[0:2]
    wua = w1u[5:133]
    ta_dst, ts_src, ts_dst, ta_src = _node_tables(
        pos_state, h, x, pos_action, u, wp, wh, wx, bs, wpa, wua)

    ea_raw = edge_a2s.shape[1]
    ea_pad = -(-ea_raw // (NWORK * CHUNK)) * (NWORK * CHUNK)
    pad = ea_pad - ea_raw

    def pad_idx(v, fill):
        return jnp.concatenate(
            [v.astype(jnp.int32), jnp.full((pad,), fill, jnp.int32)])

    src_a2d = pad_idx(edge_a2s[0], 0)
    dst_a2d = pad_idx(edge_a2s[1], N_NODE)
    src_s2d = pad_idx(edge_s2s[0], 0)
    dst_s2d = pad_idx(edge_s2s[1], N_NODE)
    dis_a = jnp.concatenate([dis_a2s, jnp.zeros((pad, 1), F32)])
    dis_s = jnp.concatenate([dis_s2s, jnp.zeros((pad, 1), F32)])

    g_as, g_ad, g_ss, g_sd = _gather4(
        src_a2d, dst_a2d, src_s2d, dst_s2d, ta_src, ta_dst, ts_src, ts_dst)

    num_a, den_a = _edge_mlp(g_as, g_ad, dis_a, w1u[4:5], w2u,
                             b2u.reshape(1, 2 * 64), w3u,
                             b3u.reshape(1, 2 * f))
    num_s, den_s = _edge_mlp(g_ss, g_sd, dis_s, w1x[4:5], w2x,
                             b2x.reshape(1, 2 * 64), w3x,
                             b3x.reshape(1, 2 * f))

    zeros = jnp.zeros((N_NODE + 8, f), F32)
    numa, dena = _scatter_add(dst_a2d, num_a, den_a, zeros)
    nums, dens = _scatter_add(dst_s2d, num_s, den_s, zeros)

    pu = params["h_updater"]
    w1f = pu["W1"]
    return _final_mlp(
        pos_state, h, x, numa, dena, nums, dens,
        w1f[0:2], w1f[2:130], w1f[130:258], w1f[258:386], w1f[386:514],
        pu["b1"].reshape(1, -1), pu["W2"], pu["b2"].reshape(1, -1),
        pu["W3"], pu["b3"].reshape(1, -1))
